# Initial kernel scaffold; baseline (speedup 1.0000x reference)
import jax
import jax.numpy as jnp
from jax.experimental import pallas as pl


def kernel(x, Ws, bs, We, be, Wg, bg, gate_bias):
    raise NotImplementedError("write your pallas kernel here")


# speedup vs baseline: 2.3245x; 2.3245x over previous
import jax
import jax.numpy as jnp
from jax.experimental import pallas as pl
from jax.experimental.pallas import tpu as pltpu

N = 2048
D = 1024
E = 8
TM = 1024


def _moe_body(x_ref, ws_ref, bs_ref, we_ref, be_ref, wg_ref, bg_ref,
              o_ref, gw_ref):
    e = pl.program_id(1)
    x = x_ref[...]

    @pl.when(e == 0)
    def _init():
        g = jax.lax.dot_general(x, wg_ref[...], (((1,), (1,)), ((), ())),
                                preferred_element_type=jnp.float32)
        g = g + bg_ref[...]
        ii = jax.lax.broadcasted_iota(jnp.int32, (TM, E), 1)
        m0 = jnp.max(g, axis=1, keepdims=True)
        i0 = jnp.min(jnp.where(g == m0, ii, E), axis=1, keepdims=True)
        g2 = jnp.where(ii == i0, -1e30, g)
        m1 = jnp.max(g2, axis=1, keepdims=True)
        i1 = jnp.min(jnp.where(g2 == m1, ii, E), axis=1, keepdims=True)
        w0 = 1.0 / (1.0 + jnp.exp(m1 - m0))
        gw_ref[...] = jnp.where(ii == i0, w0, 0.0) + jnp.where(
            ii == i1, 1.0 - w0, 0.0)
        shared = jax.lax.dot_general(x, ws_ref[...], (((1,), (1,)), ((), ())),
                                     preferred_element_type=jnp.float32)
        o_ref[...] = x + shared + bs_ref[...]

    ii = jax.lax.broadcasted_iota(jnp.int32, (TM, E), 1)
    wcol = jnp.sum(jnp.where(ii == e, gw_ref[...], 0.0), axis=1, keepdims=True)
    r = jax.lax.dot_general(x, we_ref[0], (((1,), (1,)), ((), ())),
                            preferred_element_type=jnp.float32)
    o_ref[...] += wcol * (r + be_ref[0])

    @pl.when(e == E - 1)
    def _fin():
        o_ref[...] = jnp.maximum(o_ref[...], 0.0)


def kernel(x, Ws, bs, We, be, Wg, bg, gate_bias):
    bs2 = bs.reshape(1, D)
    be3 = be.reshape(E, 1, D)
    bg2 = (bg + gate_bias).reshape(1, E)
    nt = N // TM
    return pl.pallas_call(
        _moe_body,
        grid=(nt, E),
        in_specs=[
            pl.BlockSpec((TM, D), lambda i, e: (i, 0)),
            pl.BlockSpec((D, D), lambda i, e: (0, 0)),
            pl.BlockSpec((1, D), lambda i, e: (0, 0)),
            pl.BlockSpec((1, D, D), lambda i, e: (e, 0, 0)),
            pl.BlockSpec((1, 1, D), lambda i, e: (e, 0, 0)),
            pl.BlockSpec((E, D), lambda i, e: (0, 0)),
            pl.BlockSpec((1, E), lambda i, e: (0, 0)),
        ],
        out_specs=pl.BlockSpec((TM, D), lambda i, e: (i, 0)),
        out_shape=jax.ShapeDtypeStruct((N, D), jnp.float32),
        scratch_shapes=[pltpu.VMEM((TM, E), jnp.float32)],
    )(x, Ws, bs2, We, be3, Wg, bg2)
